# baseline (speedup 1.0000x reference)
import functools

import jax
import jax.numpy as jnp
from jax import lax
from jax.experimental import pallas as pl
from jax.experimental.pallas import tpu as pltpu
from jax.experimental.pallas import tpu_sc as plsc

N = 8192
D = 1024
E = 8
H = 16

BS = 1024
BT = 256
NW = 32
CH = N // NW
RB = 32


def _sel_body(x_ref, wl_ref, bl_ref, ws1_ref, bs1_ref, ws2_ref, bs2_ref,
              ids_ref, xbf_ref, hist_ref):
    x = x_ref[...]
    lo = jnp.dot(x, wl_ref[...], preferred_element_type=jnp.float32)
    lo = lo + bl_ref[...]
    h = jnp.maximum(jnp.dot(lo, ws1_ref[...],
                            preferred_element_type=jnp.float32)
                    + bs1_ref[...], 0.0)
    logits = jnp.dot(h, ws2_ref[...],
                     preferred_element_type=jnp.float32) + bs2_ref[...]
    m = jnp.max(logits, axis=-1, keepdims=True)
    eidx = jax.lax.broadcasted_iota(jnp.int32, logits.shape, 1)
    first = jnp.min(jnp.where(logits == m, eidx, E), axis=-1)
    ids_ref[...] = first
    xbf_ref[...] = x.astype(jnp.bfloat16)
    eidx16 = jax.lax.broadcasted_iota(jnp.int32, (BS, 16), 1)
    onh = jnp.where(eidx16 == first[:, None], 1, 0)
    for c in range(BS // CH):
        hist_ref[0, c] = jnp.sum(onh[c * CH:(c + 1) * CH], axis=0)


def _selector(x, W_learner, b_learner, W_sel1, b_sel1, W_sel2, b_sel2):
    resident = lambda *shape: pl.BlockSpec(shape, lambda i: (0,) * len(shape))
    ids3 = pl.pallas_call(
        _sel_body,
        grid=(N // BS,),
        in_specs=[
            pl.BlockSpec((BS, D), lambda i: (i, 0)),
            resident(D, D),
            resident(D),
            resident(D, H),
            resident(H),
            resident(H, E),
            resident(E),
        ],
        out_specs=[
            pl.BlockSpec((BS,), lambda i: (i,)),
            pl.BlockSpec((BS, D), lambda i: (i, 0)),
            pl.BlockSpec((1, BS // CH, 16), lambda i: (i, 0, 0)),
        ],
        out_shape=[
            jax.ShapeDtypeStruct((N,), jnp.int32),
            jax.ShapeDtypeStruct((N, D), jnp.bfloat16),
            jax.ShapeDtypeStruct((N // BS, BS // CH, 16), jnp.int32),
        ],
    )(x, W_learner, b_learner, W_sel1, b_sel1, W_sel2, b_sel2)
    ids, xbf, hist = ids3
    return ids, xbf, hist.reshape(NW, 16)


def _route_body(ids_hbm, x_hbm, hist_hbm, pos_hbm, offs_hbm, xs_hbm,
                ids_v, hist_all, offs_v, pidx_v, rows_a, rows_b,
                rsem_a, rsem_b, wsem_a, wsem_b, base_v):
    nc = 2
    wid = lax.axis_index("s") * nc + lax.axis_index("c")
    base = wid * CH
    lane = lax.iota(jnp.int32, 16)
    zero16 = jnp.zeros((16,), jnp.int32)
    one16 = jnp.ones((16,), jnp.int32)
    splat = lambda s: jnp.full((16,), s, jnp.int32)

    pltpu.sync_copy(ids_hbm.at[pl.ds(base, CH)], ids_v)

    pltpu.sync_copy(hist_hbm, hist_all)
    widv = splat(wid)
    running = zero16
    total = zero16
    for w in range(NW):
        h_w = hist_all[w]
        sel = jnp.where(splat(w) < widv, one16, zero16)
        running = running + h_w * sel
        total = total + h_w
    total = jnp.where(lane < E, total, zero16)
    excl = plsc.cumsum(total) - total
    base_v[...] = excl + running

    @pl.when(wid == 0)
    def _():
        offs_v[...] = excl
        pltpu.sync_copy(offs_v, offs_hbm)

    for j in range(CH // 16):
        v = ids_v[pl.ds(j * 16, 16)]
        blane = plsc.load_gather(base_v, [v])
        pos = zero16
        newc = zero16
        for e in range(E):
            m = v == e
            mi = jnp.where(m, one16, zero16)
            c = plsc.cumsum(mi)
            pos = jnp.where(m, blane + c - one16, pos)
            cnt = plsc.all_reduce_population_count(m)
            newc = newc + jnp.where(lane == e, cnt, zero16)
        base_v[...] = base_v[...] + newc
        pidx_v[j // (RB // 16), pl.ds((j % (RB // 16)) * 16, 16)] = pos
    pltpu.sync_copy(pidx_v, pos_hbm.at[wid])

    bufs = (rows_a, rows_b)
    rsems = (rsem_a, rsem_b)
    wsems = (wsem_a, wsem_b)
    nch = CH // RB
    reads = [None] * nch
    writes = [None] * nch
    reads[0] = pltpu.async_copy(x_hbm.at[pl.ds(base, RB)], bufs[0], rsems[0])
    for c in range(nch):
        reads[c].wait()
        writes[c] = pltpu.async_copy(bufs[c % 2], xs_hbm.at[pidx_v.at[c]],
                                     wsems[c % 2])
        if c + 1 < nch:
            if c >= 1:
                writes[c - 1].wait()
            reads[c + 1] = pltpu.async_copy(
                x_hbm.at[pl.ds(base + (c + 1) * RB, RB)],
                bufs[(c + 1) % 2], rsems[(c + 1) % 2])
    writes[nch - 2].wait()
    writes[nch - 1].wait()


def _route(ids, hist, xbf):
    mesh = plsc.VectorSubcoreMesh(core_axis_name="c", subcore_axis_name="s")
    k = pl.kernel(
        _route_body,
        out_type=(
            jax.ShapeDtypeStruct((NW, CH // RB, RB), jnp.int32),
            jax.ShapeDtypeStruct((16,), jnp.int32),
            jax.ShapeDtypeStruct((N, D // 2), jnp.int32),
        ),
        mesh=mesh,
        scratch_types=[
            pltpu.VMEM((CH,), jnp.int32),
            pltpu.VMEM((NW, 16), jnp.int32),
            pltpu.VMEM((16,), jnp.int32),
            pltpu.VMEM((CH // RB, RB), jnp.int32),
            pltpu.VMEM((RB, D // 2), jnp.int32),
            pltpu.VMEM((RB, D // 2), jnp.int32),
            pltpu.SemaphoreType.DMA,
            pltpu.SemaphoreType.DMA,
            pltpu.SemaphoreType.DMA,
            pltpu.SemaphoreType.DMA,
            pltpu.VMEM((16,), jnp.int32),
        ],
        compiler_params=pltpu.CompilerParams(needs_layout_passes=False),
    )
    return k(ids, xbf, hist)


def _grouped_body(offs_ref, xs_ref, wl_ref, bl_ref, ws1_ref, bs1_ref,
                  ws2_ref, bs2_ref, wens_ref, bens_ref, out_ref):
    i = pl.program_id(0)
    r0 = i * BT
    xb = xs_ref[...]
    lo = jnp.dot(xb, wl_ref[...], preferred_element_type=jnp.float32)
    lo = lo + bl_ref[...]
    h = jnp.maximum(jnp.dot(lo, ws1_ref[...],
                            preferred_element_type=jnp.float32)
                    + bs1_ref[...], 0.0)
    logits = jnp.dot(h, ws2_ref[...],
                     preferred_element_type=jnp.float32) + bs2_ref[...]
    gate = jnp.max(logits, axis=-1, keepdims=True)

    rows = r0 + jax.lax.broadcasted_iota(jnp.int32, (BT, 1), 0)
    erow = jnp.zeros((BT, 1), jnp.int32)
    e_lo = jnp.int32(0)
    e_hi = jnp.int32(0)
    for k in range(1, E + 1):
        e_lo = e_lo + jnp.where(offs_ref[k] <= r0, 1, 0)
        e_hi = e_hi + jnp.where(offs_ref[k] <= r0 + BT - 1, 1, 0)
        erow = erow + jnp.where(rows >= offs_ref[k], 1, 0)

    eidx = jax.lax.broadcasted_iota(jnp.int32, (BT, E), 1)
    gmask = jnp.where(eidx == erow, gate, 0.0)
    lo = lo + jnp.dot(gmask, bens_ref[...],
                      preferred_element_type=jnp.float32)

    def body(e, acc):
        in_e = (rows >= offs_ref[e]) & (rows < offs_ref[e + 1])
        g = jnp.where(in_e, gate, 0.0).astype(jnp.bfloat16)
        xg = xb * g
        w = wens_ref[pl.ds(e * D, D), :]
        return acc + jnp.dot(xg, w, preferred_element_type=jnp.float32)

    out_ref[...] = lax.fori_loop(e_lo, e_hi + 1, body, lo)


def _grouped(offs, xs, W_learner, b_learner, W_sel1, b_sel1, W_sel2, b_sel2,
             W_ens_bf, b_ens):
    resident = lambda *shape: pl.BlockSpec(
        shape, lambda i, *_: (0,) * len(shape))
    grid_spec = pltpu.PrefetchScalarGridSpec(
        num_scalar_prefetch=1,
        grid=(N // BT,),
        in_specs=[
            pl.BlockSpec((BT, D), lambda i, *_: (i, 0)),
            resident(D, D),
            resident(D),
            resident(D, H),
            resident(H),
            resident(H, E),
            resident(E),
            resident(E * D, D),
            resident(E, D),
        ],
        out_specs=pl.BlockSpec((BT, D), lambda i, *_: (i, 0)),
    )
    return pl.pallas_call(
        _grouped_body,
        grid_spec=grid_spec,
        out_shape=jax.ShapeDtypeStruct((N, D), jnp.float32),
    )(offs, xs, W_learner.astype(jnp.bfloat16), b_learner, W_sel1, b_sel1,
      W_sel2, b_sel2, W_ens_bf, b_ens)


def _unsort_body(pos_hbm, outs_hbm, out_hbm, pidx_v, rows_a, rows_b,
                 rsem_a, rsem_b, wsem_a, wsem_b):
    nc = 2
    wid = lax.axis_index("s") * nc + lax.axis_index("c")
    base = wid * CH
    pltpu.sync_copy(pos_hbm.at[wid], pidx_v)
    bufs = (rows_a, rows_b)
    rsems = (rsem_a, rsem_b)
    wsems = (wsem_a, wsem_b)
    nch = CH // RB
    reads = [None] * nch
    writes = [None] * nch
    reads[0] = pltpu.async_copy(outs_hbm.at[pidx_v.at[0]], bufs[0], rsems[0])
    for c in range(nch):
        reads[c].wait()
        writes[c] = pltpu.async_copy(
            bufs[c % 2], out_hbm.at[pl.ds(base + c * RB, RB)], wsems[c % 2])
        if c + 1 < nch:
            if c >= 1:
                writes[c - 1].wait()
            reads[c + 1] = pltpu.async_copy(outs_hbm.at[pidx_v.at[c + 1]],
                                            bufs[(c + 1) % 2],
                                            rsems[(c + 1) % 2])
    writes[nch - 2].wait()
    writes[nch - 1].wait()


def _unsort(pos, outs):
    mesh = plsc.VectorSubcoreMesh(core_axis_name="c", subcore_axis_name="s")
    k = pl.kernel(
        _unsort_body,
        out_type=jax.ShapeDtypeStruct((N, D), jnp.float32),
        mesh=mesh,
        scratch_types=[
            pltpu.VMEM((CH // RB, RB), jnp.int32),
            pltpu.VMEM((RB, D), jnp.float32),
            pltpu.VMEM((RB, D), jnp.float32),
            pltpu.SemaphoreType.DMA,
            pltpu.SemaphoreType.DMA,
            pltpu.SemaphoreType.DMA,
            pltpu.SemaphoreType.DMA,
        ],
        compiler_params=pltpu.CompilerParams(needs_layout_passes=False),
    )
    return k(pos, outs)


def kernel(x, W_learner, b_learner, W_sel1, b_sel1, W_sel2, b_sel2, W_ens, b_ens):
    ids, xbf, hist = _selector(x, W_learner, b_learner, W_sel1, b_sel1,
                               W_sel2, b_sel2)
    xpacked = lax.bitcast_convert_type(xbf.reshape(N, D // 2, 2), jnp.int32)
    pos, offs, xs_packed = _route(ids, hist, xpacked)
    xs = lax.bitcast_convert_type(xs_packed, jnp.bfloat16).reshape(N, D)
    outs = _grouped(offs, xs, W_learner, b_learner, W_sel1, b_sel1, W_sel2,
                    b_sel2, W_ens.astype(jnp.bfloat16).reshape(E * D, D),
                    b_ens)
    return _unsort(pos, outs)


# speedup vs baseline: 2.2358x; 2.2358x over previous
import functools

import jax
import jax.numpy as jnp
from jax import lax
from jax.experimental import pallas as pl
from jax.experimental.pallas import tpu as pltpu
from jax.experimental.pallas import tpu_sc as plsc

N = 8192
D = 1024
E = 8
H = 16

BS = 1024
BT = 256
NW = 32
CH = N // NW
RB = 32


def _sel_body(x_ref, wl_ref, bl_ref, ws1_ref, bs1_ref, ws2_ref, bs2_ref,
              ids_ref, hist_ref):
    x = x_ref[...]
    lo = jnp.dot(x, wl_ref[...], preferred_element_type=jnp.float32)
    lo = lo + bl_ref[...]
    h = jnp.maximum(jnp.dot(lo, ws1_ref[...],
                            preferred_element_type=jnp.float32)
                    + bs1_ref[...], 0.0)
    logits = jnp.dot(h, ws2_ref[...],
                     preferred_element_type=jnp.float32) + bs2_ref[...]
    m = jnp.max(logits, axis=-1, keepdims=True)
    eidx = jax.lax.broadcasted_iota(jnp.int32, logits.shape, 1)
    first = jnp.min(jnp.where(logits == m, eidx, E), axis=-1)
    ids_ref[...] = first
    eidx16 = jax.lax.broadcasted_iota(jnp.int32, (BS, 16), 1)
    onh = jnp.where(eidx16 == first[:, None], 1, 0)
    for c in range(BS // CH):
        hist_ref[0, c] = jnp.sum(onh[c * CH:(c + 1) * CH], axis=0)


def _selector(x, W_learner, b_learner, W_sel1, b_sel1, W_sel2, b_sel2):
    resident = lambda *shape: pl.BlockSpec(shape, lambda i: (0,) * len(shape))
    ids3 = pl.pallas_call(
        _sel_body,
        grid=(N // BS,),
        in_specs=[
            pl.BlockSpec((BS, D), lambda i: (i, 0)),
            resident(D, D),
            resident(D),
            resident(D, H),
            resident(H),
            resident(H, E),
            resident(E),
        ],
        out_specs=[
            pl.BlockSpec((BS,), lambda i: (i,)),
            pl.BlockSpec((1, BS // CH, 16), lambda i: (i, 0, 0)),
        ],
        out_shape=[
            jax.ShapeDtypeStruct((N,), jnp.int32),
            jax.ShapeDtypeStruct((N // BS, BS // CH, 16), jnp.int32),
        ],
    )(x, W_learner, b_learner, W_sel1, b_sel1, W_sel2, b_sel2)
    ids, hist = ids3
    return ids, hist.reshape(NW, 16)


def _route_body(ids_hbm, x_hbm, hist_hbm, pos_hbm, offs_hbm, xs_hbm,
                ids_v, hist_all, offs_v, pidx_v, rows_a, rows_b,
                rsem_a, rsem_b, wsem_a, wsem_b, base_v):
    nc = 2
    wid = lax.axis_index("s") * nc + lax.axis_index("c")
    base = wid * CH
    lane = lax.iota(jnp.int32, 16)
    zero16 = jnp.zeros((16,), jnp.int32)
    one16 = jnp.ones((16,), jnp.int32)
    splat = lambda s: jnp.full((16,), s, jnp.int32)

    pltpu.sync_copy(ids_hbm.at[pl.ds(base, CH)], ids_v)

    pltpu.sync_copy(hist_hbm, hist_all)
    widv = splat(wid)
    running = zero16
    total = zero16
    for w in range(NW):
        h_w = hist_all[w]
        sel = jnp.where(splat(w) < widv, one16, zero16)
        running = running + h_w * sel
        total = total + h_w
    total = jnp.where(lane < E, total, zero16)
    excl = plsc.cumsum(total) - total
    base_v[...] = excl + running

    @pl.when(wid == 0)
    def _():
        offs_v[...] = excl
        pltpu.sync_copy(offs_v, offs_hbm)

    for j in range(CH // 16):
        v = ids_v[pl.ds(j * 16, 16)]
        blane = plsc.load_gather(base_v, [v])
        pos = zero16
        newc = zero16
        for e in range(E):
            m = v == e
            mi = jnp.where(m, one16, zero16)
            c = plsc.cumsum(mi)
            pos = jnp.where(m, blane + c - one16, pos)
            cnt = plsc.all_reduce_population_count(m)
            newc = newc + jnp.where(lane == e, cnt, zero16)
        base_v[...] = base_v[...] + newc
        pidx_v[j // (RB // 16), pl.ds((j % (RB // 16)) * 16, 16)] = pos
    pltpu.sync_copy(pidx_v, pos_hbm.at[wid])

    bufs = (rows_a, rows_b)
    rsems = (rsem_a, rsem_b)
    wsems = (wsem_a, wsem_b)
    nch = CH // RB
    reads = [None] * nch
    writes = [None] * nch
    reads[0] = pltpu.async_copy(x_hbm.at[pl.ds(base, RB)], bufs[0], rsems[0])
    for c in range(nch):
        reads[c].wait()
        writes[c] = pltpu.async_copy(bufs[c % 2], xs_hbm.at[pidx_v.at[c]],
                                     wsems[c % 2])
        if c + 1 < nch:
            if c >= 1:
                writes[c - 1].wait()
            reads[c + 1] = pltpu.async_copy(
                x_hbm.at[pl.ds(base + (c + 1) * RB, RB)],
                bufs[(c + 1) % 2], rsems[(c + 1) % 2])
    writes[nch - 2].wait()
    writes[nch - 1].wait()


def _route(ids, hist, xbf):
    mesh = plsc.VectorSubcoreMesh(core_axis_name="c", subcore_axis_name="s")
    k = pl.kernel(
        _route_body,
        out_type=(
            jax.ShapeDtypeStruct((NW, CH // RB, RB), jnp.int32),
            jax.ShapeDtypeStruct((16,), jnp.int32),
            jax.ShapeDtypeStruct((N, D), jnp.float32),
        ),
        mesh=mesh,
        scratch_types=[
            pltpu.VMEM((CH,), jnp.int32),
            pltpu.VMEM((NW, 16), jnp.int32),
            pltpu.VMEM((16,), jnp.int32),
            pltpu.VMEM((CH // RB, RB), jnp.int32),
            pltpu.VMEM((RB, D), jnp.float32),
            pltpu.VMEM((RB, D), jnp.float32),
            pltpu.SemaphoreType.DMA,
            pltpu.SemaphoreType.DMA,
            pltpu.SemaphoreType.DMA,
            pltpu.SemaphoreType.DMA,
            pltpu.VMEM((16,), jnp.int32),
        ],
        compiler_params=pltpu.CompilerParams(needs_layout_passes=False),
    )
    return k(ids, xbf, hist)


def _grouped_body(offs_ref, xs_ref, wl_ref, bl_ref, ws1_ref, bs1_ref,
                  ws2_ref, bs2_ref, wens_ref, bens_ref, out_ref):
    i = pl.program_id(0)
    r0 = i * BT
    x = xs_ref[...]
    xb = x.astype(jnp.bfloat16)
    lo = jnp.dot(xb, wl_ref[...], preferred_element_type=jnp.float32)
    lo = lo + bl_ref[...]
    h = jnp.maximum(jnp.dot(lo, ws1_ref[...],
                            preferred_element_type=jnp.float32)
                    + bs1_ref[...], 0.0)
    logits = jnp.dot(h, ws2_ref[...],
                     preferred_element_type=jnp.float32) + bs2_ref[...]
    gate = jnp.max(logits, axis=-1, keepdims=True)

    rows = r0 + jax.lax.broadcasted_iota(jnp.int32, (BT, 1), 0)
    erow = jnp.zeros((BT, 1), jnp.int32)
    e_lo = jnp.int32(0)
    e_hi = jnp.int32(0)
    for k in range(1, E + 1):
        e_lo = e_lo + jnp.where(offs_ref[k] <= r0, 1, 0)
        e_hi = e_hi + jnp.where(offs_ref[k] <= r0 + BT - 1, 1, 0)
        erow = erow + jnp.where(rows >= offs_ref[k], 1, 0)

    eidx = jax.lax.broadcasted_iota(jnp.int32, (BT, E), 1)
    gmask = jnp.where(eidx == erow, gate, 0.0)
    lo = lo + jnp.dot(gmask, bens_ref[...],
                      preferred_element_type=jnp.float32)

    def body(e, acc):
        in_e = (rows >= offs_ref[e]) & (rows < offs_ref[e + 1])
        g = jnp.where(in_e, gate, 0.0)
        xg = (x * g).astype(jnp.bfloat16)
        w = wens_ref[pl.ds(e * D, D), :]
        return acc + jnp.dot(xg, w, preferred_element_type=jnp.float32)

    out_ref[...] = lax.fori_loop(e_lo, e_hi + 1, body, lo)


def _grouped(offs, xs, W_learner, b_learner, W_sel1, b_sel1, W_sel2, b_sel2,
             W_ens_bf, b_ens):
    resident = lambda *shape: pl.BlockSpec(
        shape, lambda i, *_: (0,) * len(shape))
    grid_spec = pltpu.PrefetchScalarGridSpec(
        num_scalar_prefetch=1,
        grid=(N // BT,),
        in_specs=[
            pl.BlockSpec((BT, D), lambda i, *_: (i, 0)),
            resident(D, D),
            resident(D),
            resident(D, H),
            resident(H),
            resident(H, E),
            resident(E),
            resident(E * D, D),
            resident(E, D),
        ],
        out_specs=pl.BlockSpec((BT, D), lambda i, *_: (i, 0)),
    )
    return pl.pallas_call(
        _grouped_body,
        grid_spec=grid_spec,
        out_shape=jax.ShapeDtypeStruct((N, D), jnp.float32),
    )(offs, xs, W_learner.astype(jnp.bfloat16), b_learner, W_sel1, b_sel1,
      W_sel2, b_sel2, W_ens_bf, b_ens)


def _unsort_body(pos_hbm, outs_hbm, out_hbm, pidx_v, rows_a, rows_b,
                 rsem_a, rsem_b, wsem_a, wsem_b):
    nc = 2
    wid = lax.axis_index("s") * nc + lax.axis_index("c")
    base = wid * CH
    pltpu.sync_copy(pos_hbm.at[wid], pidx_v)
    bufs = (rows_a, rows_b)
    rsems = (rsem_a, rsem_b)
    wsems = (wsem_a, wsem_b)
    nch = CH // RB
    reads = [None] * nch
    writes = [None] * nch
    reads[0] = pltpu.async_copy(outs_hbm.at[pidx_v.at[0]], bufs[0], rsems[0])
    for c in range(nch):
        reads[c].wait()
        writes[c] = pltpu.async_copy(
            bufs[c % 2], out_hbm.at[pl.ds(base + c * RB, RB)], wsems[c % 2])
        if c + 1 < nch:
            if c >= 1:
                writes[c - 1].wait()
            reads[c + 1] = pltpu.async_copy(outs_hbm.at[pidx_v.at[c + 1]],
                                            bufs[(c + 1) % 2],
                                            rsems[(c + 1) % 2])
    writes[nch - 2].wait()
    writes[nch - 1].wait()


def _unsort(pos, outs):
    mesh = plsc.VectorSubcoreMesh(core_axis_name="c", subcore_axis_name="s")
    k = pl.kernel(
        _unsort_body,
        out_type=jax.ShapeDtypeStruct((N, D), jnp.float32),
        mesh=mesh,
        scratch_types=[
            pltpu.VMEM((CH // RB, RB), jnp.int32),
            pltpu.VMEM((RB, D), jnp.float32),
            pltpu.VMEM((RB, D), jnp.float32),
            pltpu.SemaphoreType.DMA,
            pltpu.SemaphoreType.DMA,
            pltpu.SemaphoreType.DMA,
            pltpu.SemaphoreType.DMA,
        ],
        compiler_params=pltpu.CompilerParams(needs_layout_passes=False),
    )
    return k(pos, outs)


def kernel(x, W_learner, b_learner, W_sel1, b_sel1, W_sel2, b_sel2, W_ens, b_ens):
    ids, hist = _selector(x, W_learner, b_learner, W_sel1, b_sel1, W_sel2,
                          b_sel2)
    pos, offs, xs = _route(ids, hist, x)
    outs = _grouped(offs, xs, W_learner, b_learner, W_sel1, b_sel1, W_sel2,
                    b_sel2, W_ens.astype(jnp.bfloat16).reshape(E * D, D),
                    b_ens)
    return _unsort(pos, outs)
